# baseline (speedup 1.0000x reference)
import jax
import jax.numpy as jnp
from jax.experimental import pallas as pl
from jax.experimental.pallas import tpu as pltpu
from jax.experimental.pallas import tpu_sc as plsc

_TOP_K = 2
_LOAD_BALANCE_ALPHA = 0.01
_Z_LOSS_ALPHA = 0.0001
_SC_NC, _SC_NS, _SC_L = 2, 16, 16
_NW = _SC_NC * _SC_NS

_INTERPRET = False


def _tc_tile(x_ref, wt_ref, bias_ref, rtemp_ref, pk_ref, pp_ref, pz_ref):
    logits = jnp.dot(x_ref[...], wt_ref[...],
                     preferred_element_type=jnp.float32)
    lt = (logits.T + bias_ref[...]) * rtemp_ref[0, 0]
    c = jnp.max(lt)
    ex = jnp.exp(lt - c)
    se = jnp.sum(ex, axis=0, keepdims=True)
    num_e = ex.shape[0]
    eidx = jax.lax.broadcasted_iota(jnp.int32, ex.shape, 0)
    pk_ref[...] = (jax.lax.bitcast_convert_type(ex, jnp.int32) & ~63) \
        | ((num_e - 1) - eidx)
    lse = c + jnp.log(se)
    pp_ref[...] = jnp.sum(ex * (1.0 / se), axis=1, keepdims=True).T[None]
    pz_ref[...] = jnp.broadcast_to(jnp.sum(lse * lse), pz_ref.shape)


def _sc_route(pk_hbm, ts_hbm, ti_hbm, pk_v, ts_v, ti_v):
    wid = jax.lax.axis_index("s") * _SC_NC + jax.lax.axis_index("c")
    E, CH = pk_v.shape
    base = wid * CH
    pltpu.sync_copy(pk_hbm.at[:, pl.ds(base, CH)], pk_v)

    def group(g, carry):
        off = g * _SC_L
        r1 = pk_v[0, pl.ds(off, _SC_L)]
        r2 = jnp.zeros((_SC_L,), jnp.int32)
        for e in range(1, E):
            pe = pk_v[e, pl.ds(off, _SC_L)]
            lo = jnp.minimum(r1, pe)
            r1 = jnp.maximum(r1, pe)
            r2 = jnp.maximum(r2, lo)
        i1 = (E - 1) - (r1 & 63)
        i2 = (E - 1) - (r2 & 63)
        v1 = jax.lax.bitcast_convert_type(r1 & ~63, jnp.float32)
        v2 = jax.lax.bitcast_convert_type(r2 & ~63, jnp.float32)
        rden = 1.0 / (v1 + v2)
        ts_v[0, pl.ds(off, _SC_L)] = v1 * rden
        ts_v[1, pl.ds(off, _SC_L)] = v2 * rden
        ti_v[0, pl.ds(off, _SC_L)] = i1
        ti_v[1, pl.ds(off, _SC_L)] = i2
        return carry

    jax.lax.fori_loop(0, CH // _SC_L, group, 0)
    pltpu.sync_copy(ts_v, ts_hbm.at[:, pl.ds(base, CH)])
    pltpu.sync_copy(ti_v, ti_hbm.at[:, pl.ds(base, CH)])


def kernel(x, W, expert_bias, temperature):
    B, S, H = x.shape
    E = W.shape[0]
    T = B * S
    x_flat = x.reshape(T, H)
    rtemp = (1.0 / jnp.asarray(temperature, jnp.float32)).reshape(1, 1)
    wt = W.T
    bias = expert_bias.reshape(E, 1)
    TT = 4096
    G = T // TT
    pk, pp, pz = pl.pallas_call(
        _tc_tile,
        grid=(G,),
        in_specs=[
            pl.BlockSpec((TT, H), lambda i: (i, 0)),
            pl.BlockSpec((H, E), lambda i: (0, 0)),
            pl.BlockSpec((E, 1), lambda i: (0, 0)),
            pl.BlockSpec((1, 1), lambda i: (0, 0)),
        ],
        out_specs=[
            pl.BlockSpec((E, TT), lambda i: (0, i)),
            pl.BlockSpec((1, 1, E), lambda i: (i, 0, 0)),
            pl.BlockSpec((1, 1, E), lambda i: (i, 0, 0)),
        ],
        out_shape=[
            jax.ShapeDtypeStruct((E, T), jnp.int32),
            jax.ShapeDtypeStruct((G, 1, E), jnp.float32),
            jax.ShapeDtypeStruct((G, 1, E), jnp.float32),
        ],
        compiler_params=pltpu.CompilerParams(
            dimension_semantics=("parallel",)),
        interpret=_INTERPRET,
    )(x_flat, wt, bias, rtemp)

    CH = T // _NW
    ts, ti = pl.kernel(
        _sc_route,
        out_type=[
            jax.ShapeDtypeStruct((_TOP_K, T), jnp.float32),
            jax.ShapeDtypeStruct((_TOP_K, T), jnp.int32),
        ],
        mesh=plsc.VectorSubcoreMesh(
            core_axis_name="c", subcore_axis_name="s",
            num_cores=_SC_NC, num_subcores=_SC_NS),
        scratch_types=[
            pltpu.VMEM((E, CH), jnp.int32),
            pltpu.VMEM((_TOP_K, CH), jnp.float32),
            pltpu.VMEM((_TOP_K, CH), jnp.int32),
        ],
        interpret=_INTERPRET,
    )(pk)

    ts = ts.T
    ti = ti.T
    one_hot = (ti[:, :, None] == jnp.arange(E)[None, None, :])
    f = one_hot.sum(axis=(0, 1)).astype(jnp.float32) / T
    P = jnp.sum(pp[:, 0, :], axis=0) / T
    z = jnp.sum(pz[:, 0, 0]) / T
    aux = _LOAD_BALANCE_ALPHA * E * jnp.sum(f * P)
    total = aux + _Z_LOSS_ALPHA * z
    return ts, ti, total


# speedup vs baseline: 1.5394x; 1.5394x over previous
import jax
import jax.numpy as jnp
from jax.experimental import pallas as pl
from jax.experimental.pallas import tpu as pltpu

_TOP_K = 2
_LOAD_BALANCE_ALPHA = 0.01
_Z_LOSS_ALPHA = 0.0001


def _gate_tile(x_ref, wt_ref, bias_ref, rtemp_ref, ts_ref, ti_ref,
               pf_ref, pp_ref, pz_ref):
    logits = jnp.dot(x_ref[...], wt_ref[...],
                     preferred_element_type=jnp.float32)
    lt = (logits.T + bias_ref[...]) * rtemp_ref[0, 0]
    c = jnp.max(lt)
    ex = jnp.exp(lt - c)
    se = jnp.sum(ex, axis=0, keepdims=True)
    num_e = ex.shape[0]
    eidx = jax.lax.broadcasted_iota(jnp.int32, ex.shape, 0)
    pack = (jax.lax.bitcast_convert_type(ex, jnp.int32) & ~63) \
        | ((num_e - 1) - eidx)
    r1 = jnp.max(pack, axis=0, keepdims=True)
    m1 = pack == r1
    r2 = jnp.max(jnp.where(m1, 0, pack), axis=0, keepdims=True)
    i1 = (num_e - 1) - (r1 & 63)
    i2 = (num_e - 1) - (r2 & 63)
    v1 = jax.lax.bitcast_convert_type(r1 & ~63, jnp.float32)
    v2 = jax.lax.bitcast_convert_type(r2 & ~63, jnp.float32)
    rden = 1.0 / (v1 + v2)
    ts_ref[...] = jnp.concatenate([v1 * rden, v2 * rden], axis=0)
    ti_ref[...] = jnp.concatenate([i1, i2], axis=0)
    hits = m1.astype(jnp.float32) + (pack == r2).astype(jnp.float32)
    lse = c + jnp.log(se)
    pf_ref[...] = jnp.sum(hits, axis=1, keepdims=True).T[None]
    pp_ref[...] = jnp.sum(ex * (1.0 / se), axis=1, keepdims=True).T[None]
    pz_ref[...] = jnp.broadcast_to(jnp.sum(lse * lse), pz_ref.shape)


def kernel(x, W, expert_bias, temperature):
    B, S, H = x.shape
    E = W.shape[0]
    T = B * S
    x_flat = x.reshape(T, H)
    rtemp = (1.0 / jnp.asarray(temperature, jnp.float32)).reshape(1, 1)
    wt = W.T
    bias = expert_bias.reshape(E, 1)
    TT = 4096
    G = T // TT
    ts, ti, pf, pp, pz = pl.pallas_call(
        _gate_tile,
        grid=(G,),
        in_specs=[
            pl.BlockSpec((TT, H), lambda i: (i, 0)),
            pl.BlockSpec((H, E), lambda i: (0, 0)),
            pl.BlockSpec((E, 1), lambda i: (0, 0)),
            pl.BlockSpec((1, 1), lambda i: (0, 0)),
        ],
        out_specs=[
            pl.BlockSpec((_TOP_K, TT), lambda i: (0, i)),
            pl.BlockSpec((_TOP_K, TT), lambda i: (0, i)),
            pl.BlockSpec((1, 1, E), lambda i: (i, 0, 0)),
            pl.BlockSpec((1, 1, E), lambda i: (i, 0, 0)),
            pl.BlockSpec((1, 1, E), lambda i: (i, 0, 0)),
        ],
        out_shape=[
            jax.ShapeDtypeStruct((_TOP_K, T), jnp.float32),
            jax.ShapeDtypeStruct((_TOP_K, T), jnp.int32),
            jax.ShapeDtypeStruct((G, 1, E), jnp.float32),
            jax.ShapeDtypeStruct((G, 1, E), jnp.float32),
            jax.ShapeDtypeStruct((G, 1, E), jnp.float32),
        ],
        compiler_params=pltpu.CompilerParams(
            dimension_semantics=("parallel",)),
    )(x_flat, wt, bias, rtemp)
    ts = ts.T
    ti = ti.T
    f = jnp.sum(pf[:, 0, :], axis=0) / T
    P = jnp.sum(pp[:, 0, :], axis=0) / T
    z = jnp.sum(pz[:, 0, 0]) / T
    aux = _LOAD_BALANCE_ALPHA * E * jnp.sum(f * P)
    total = aux + _Z_LOSS_ALPHA * z
    return ts, ti, total


# speedup vs baseline: 1.5609x; 1.0140x over previous
import jax
import jax.numpy as jnp
from jax.experimental import pallas as pl
from jax.experimental.pallas import tpu as pltpu

_TOP_K = 2
_LOAD_BALANCE_ALPHA = 0.01
_Z_LOSS_ALPHA = 0.0001


def _gate_tile(x_ref, wt_ref, bias_ref, rtemp_ref, ts_ref, ti_ref,
               part_ref):
    logits = jnp.dot(x_ref[...], wt_ref[...],
                     preferred_element_type=jnp.float32)
    lt = (logits.T + bias_ref[...]) * rtemp_ref[0, 0]
    c = jnp.max(lt)
    ex = jnp.exp(lt - c)
    se = jnp.sum(ex, axis=0, keepdims=True)
    num_e = ex.shape[0]
    eidx = jax.lax.broadcasted_iota(jnp.int32, ex.shape, 0)
    pack = (jax.lax.bitcast_convert_type(ex, jnp.int32) & ~63) \
        | ((num_e - 1) - eidx)
    r1 = jnp.max(pack, axis=0, keepdims=True)
    m1 = pack == r1
    r2 = jnp.max(jnp.where(m1, 0, pack), axis=0, keepdims=True)
    i1 = (num_e - 1) - (r1 & 63)
    i2 = (num_e - 1) - (r2 & 63)
    v1 = jax.lax.bitcast_convert_type(r1 & ~63, jnp.float32)
    v2 = jax.lax.bitcast_convert_type(r2 & ~63, jnp.float32)
    rden = 1.0 / (v1 + v2)
    ts_ref[...] = jnp.concatenate([v1 * rden, v2 * rden], axis=0)
    ti_ref[...] = jnp.concatenate([i1, i2], axis=0)
    hits = m1.astype(jnp.float32) + (pack == r2).astype(jnp.float32)
    lse = c + jnp.log(se)
    pf = jnp.sum(hits, axis=1, keepdims=True).T
    pp = jnp.sum(ex * (1.0 / se), axis=1, keepdims=True).T
    pz = jnp.broadcast_to(jnp.sum(lse * lse), pf.shape)
    part_ref[...] = jnp.concatenate([pf, pp, pz], axis=0)[None]


def kernel(x, W, expert_bias, temperature):
    B, S, H = x.shape
    E = W.shape[0]
    T = B * S
    x_flat = x.reshape(T, H)
    rtemp = (1.0 / jnp.asarray(temperature, jnp.float32)).reshape(1, 1)
    wt = W.T
    bias = expert_bias.reshape(E, 1)
    TT = 4096
    G = T // TT
    ts, ti, part = pl.pallas_call(
        _gate_tile,
        grid=(G,),
        in_specs=[
            pl.BlockSpec((TT, H), lambda i: (i, 0)),
            pl.BlockSpec((H, E), lambda i: (0, 0)),
            pl.BlockSpec((E, 1), lambda i: (0, 0)),
            pl.BlockSpec((1, 1), lambda i: (0, 0)),
        ],
        out_specs=[
            pl.BlockSpec((_TOP_K, TT), lambda i: (0, i)),
            pl.BlockSpec((_TOP_K, TT), lambda i: (0, i)),
            pl.BlockSpec((1, 3, E), lambda i: (i, 0, 0)),
        ],
        out_shape=[
            jax.ShapeDtypeStruct((_TOP_K, T), jnp.float32),
            jax.ShapeDtypeStruct((_TOP_K, T), jnp.int32),
            jax.ShapeDtypeStruct((G, 3, E), jnp.float32),
        ],
        compiler_params=pltpu.CompilerParams(
            dimension_semantics=("parallel",)),
    )(x_flat, wt, bias, rtemp)
    ts = ts.T
    ti = ti.T
    f = jnp.sum(part[:, 0, :], axis=0) / T
    P = jnp.sum(part[:, 1, :], axis=0) / T
    z = jnp.sum(part[:, 2, 0]) / T
    aux = _LOAD_BALANCE_ALPHA * E * jnp.sum(f * P)
    total = aux + _Z_LOSS_ALPHA * z
    return ts, ti, total
